# Initial kernel scaffold; baseline (speedup 1.0000x reference)
import jax
import jax.numpy as jnp
from jax.experimental import pallas as pl


def kernel(x, edge_index, attention, conv_z_w, conv_z_b, lin_z_w, lin_z_b, conv_r_w, conv_r_b, lin_r_w, lin_r_b, conv_h_w, conv_h_b, lin_h_w, lin_h_b, out_w, out_b):
    raise NotImplementedError("write your pallas kernel here")


# speedup vs baseline: 198.3516x; 198.3516x over previous
import functools
import jax
import jax.numpy as jnp
from jax import lax
from jax.experimental import pallas as pl
from jax.experimental.pallas import tpu as pltpu
from jax.experimental.pallas import tpu_sc as plsc

N = 10000
E = 320000
T = 12
F_IN = 128
F_OUT = 3
D = 80
NC, NS = 2, 16
NW = NC * NS
EPT = E // NW
C = 80
NCH = EPT // C
EPT_DEG = E // NS
RPT = 640
RPT_LAST = N - RPT * (NS - 1)

@functools.cache
def _sc_kernels():
    mesh = plsc.VectorSubcoreMesh(core_axis_name="c", subcore_axis_name="s",
                                  num_cores=NC, num_subcores=NS)
    deg = functools.partial(
        pl.kernel,
        out_type=jax.ShapeDtypeStruct((NS, N), jnp.float32),
        mesh=mesh,
        scratch_types=[
            pltpu.VMEM((EPT_DEG,), jnp.int32),
            pltpu.VMEM((N,), jnp.float32),
        ],
        compiler_params=pltpu.CompilerParams(needs_layout_passes=False),
    )(_deg_body)
    edge = functools.partial(
        pl.kernel,
        out_type=jax.ShapeDtypeStruct((NC, N, D), jnp.float32),
        mesh=mesh,
        scratch_types=[
            pltpu.VMEM((NCH, C), jnp.int32),
            pltpu.VMEM((NCH, C), jnp.int32),
            pltpu.VMEM((C, D), jnp.float32),
            pltpu.VMEM((C, D), jnp.float32),
            pltpu.VMEM_SHARED((N, D), jnp.float32),
            pltpu.SemaphoreType.DMA,
            pltpu.SemaphoreType.DMA,
        ],
        compiler_params=pltpu.CompilerParams(use_tc_tiling_on_sc=False),
    )(_edge_body)
    return deg, edge


def _deg_body(dst_hbm, out_hbm, idx_v, acc_v):
    c = lax.axis_index("c")
    s = lax.axis_index("s")

    @pl.when(c == 0)
    def _():
        def zero_body(i, carry):
            acc_v[pl.ds(i * 16, 16)] = jnp.zeros((16,), jnp.float32)
            return carry

        lax.fori_loop(0, N // 16, zero_body, 0)

        pltpu.sync_copy(dst_hbm.at[pl.ds(s * EPT_DEG, EPT_DEG)], idx_v)

        ones = jnp.ones((16,), jnp.float32)

        def add_body(i, carry):
            iv = idx_v[pl.ds(i * 16, 16)]
            plsc.addupdate_scatter(acc_v, [iv], ones)
            return carry

        lax.fori_loop(0, EPT_DEG // 16, add_body, 0)

        pltpu.sync_copy(acc_v, out_hbm.at[s])


def _edge_body(src_hbm, dst_hbm, xws_hbm, zeros_hbm, out_hbm,
               src_v, dst_v, rows_a, rows_b, s_sh, sem_a, sem_b):
    c = lax.axis_index("c")
    s = lax.axis_index("s")
    wid = c * NS + s

    pltpu.sync_copy(src_hbm.at[wid], src_v)
    pltpu.sync_copy(dst_hbm.at[wid], dst_v)

    @pl.when(s < NS - 1)
    def _():
        pltpu.sync_copy(zeros_hbm.at[pl.ds(s * RPT, RPT)],
                        s_sh.at[pl.ds(s * RPT, RPT)])

    @pl.when(s == NS - 1)
    def _():
        pltpu.sync_copy(zeros_hbm.at[pl.ds((NS - 1) * RPT, RPT_LAST)],
                        s_sh.at[pl.ds((NS - 1) * RPT, RPT_LAST)])
    pltpu.async_copy(xws_hbm.at[src_v.at[0]], rows_a, sem_a)
    plsc.subcore_barrier()

    def step(j, rows_cur, sem_cur, rows_nxt, sem_nxt):
        @pl.when(j + 1 < NCH)
        def _():
            pltpu.async_copy(xws_hbm.at[src_v.at[j + 1]], rows_nxt, sem_nxt)

        pltpu.make_async_copy(xws_hbm.at[src_v.at[j]], rows_cur, sem_cur).wait()
        pltpu.sync_copy(rows_cur, s_sh.at[dst_v.at[j]], add=True)

    def body(i, carry):
        j = i * 2
        step(j, rows_a, sem_a, rows_b, sem_b)
        step(j + 1, rows_b, sem_b, rows_a, sem_a)
        return carry

    lax.fori_loop(0, NCH // 2, body, 0)
    if NCH % 2:
        step(NCH - 1, rows_a, sem_a, rows_b, sem_b)

    plsc.subcore_barrier()

    @pl.when(s < NS - 1)
    def _():
        pltpu.sync_copy(s_sh.at[pl.ds(s * RPT, RPT)],
                        out_hbm.at[c, pl.ds(s * RPT, RPT)])

    @pl.when(s == NS - 1)
    def _():
        pltpu.sync_copy(s_sh.at[pl.ds((NS - 1) * RPT, RPT_LAST)],
                        out_hbm.at[c, pl.ds((NS - 1) * RPT, RPT_LAST)])


def _xws_body(xf_ref, wb_ref, degt_ref, o_ref):
    deg = jnp.sum(degt_ref[...], axis=1, keepdims=True) + 1.0
    dinv = lax.rsqrt(deg)
    xw = jnp.dot(xf_ref[...], wb_ref[...], preferred_element_type=jnp.float32)
    o_ref[...] = xw * dinv


def _xws_call(xf, wb, degt, bn):
    grid = (N // bn,)
    return pl.pallas_call(
        _xws_body,
        grid=grid,
        in_specs=[
            pl.BlockSpec((bn, F_IN * T), lambda i: (i, 0)),
            pl.BlockSpec((F_IN * T, D), lambda i: (0, 0)),
            pl.BlockSpec((bn, NS), lambda i: (i, 0)),
        ],
        out_specs=pl.BlockSpec((bn, D), lambda i: (i, 0)),
        out_shape=jax.ShapeDtypeStruct((N, D), jnp.float32),
    )(xf, wb, degt)


def _post_body(s_ref, xws_ref, degt_ref, p_ref, bvec_ref, q_ref, ow_ref,
               ob_ref, o_ref):
    deg = jnp.sum(degt_ref[...], axis=1, keepdims=True) + 1.0
    dinv = lax.rsqrt(deg)
    s = s_ref[...]
    agg = (s[0] + s[1] + xws_ref[...]) * dinv
    y = jnp.dot(agg, p_ref[...], preferred_element_type=jnp.float32)
    y = y + bvec_ref[...]
    zs = jax.nn.sigmoid(y[:, 0:36])
    th = jnp.tanh(y[:, 36:72])
    ht = (1.0 - zs) * th
    hacc = jnp.dot(ht, q_ref[...], preferred_element_type=jnp.float32)
    o_ref[...] = (jnp.dot(jnp.maximum(hacc, 0.0), ow_ref[...],
                          preferred_element_type=jnp.float32) + ob_ref[...])


def _post_call(s2, xws, degt, p, bvec, q, ow, ob, bn):
    grid = (N // bn,)
    return pl.pallas_call(
        _post_body,
        grid=grid,
        in_specs=[
            pl.BlockSpec((NC, bn, D), lambda i: (0, i, 0)),
            pl.BlockSpec((bn, D), lambda i: (i, 0)),
            pl.BlockSpec((bn, NS), lambda i: (i, 0)),
            pl.BlockSpec((D, D), lambda i: (0, 0)),
            pl.BlockSpec((1, D), lambda i: (0, 0)),
            pl.BlockSpec((36, F_OUT), lambda i: (0, 0)),
            pl.BlockSpec((F_OUT, T), lambda i: (0, 0)),
            pl.BlockSpec((1, T), lambda i: (0, 0)),
        ],
        out_specs=pl.BlockSpec((bn, T), lambda i: (i, 0)),
        out_shape=jax.ShapeDtypeStruct((N, T), jnp.float32),
    )(s2, xws, degt, p, bvec, q, ow, ob)


@jax.jit
def kernel(x, edge_index, attention, conv_z_w, conv_z_b, lin_z_w, lin_z_b,
           conv_r_w, conv_r_b, lin_r_w, lin_r_b, conv_h_w, conv_h_b,
           lin_h_w, lin_h_b, out_w, out_b):
    eye_t = jnp.eye(T, dtype=jnp.float32)
    w6 = jnp.concatenate([conv_z_w, conv_h_w], axis=1)
    wb = jnp.einsum("fk,ts->ftsk", w6, eye_t).reshape(F_IN * T, T * 6)
    wb = jnp.pad(wb, ((0, 0), (0, D - T * 6)))

    lz = lin_z_w[:F_OUT]
    lh = lin_h_w[:F_OUT]
    lz6 = jnp.concatenate([lz, jnp.zeros((F_OUT, F_OUT), jnp.float32)], 0)
    lh6 = jnp.concatenate([jnp.zeros((F_OUT, F_OUT), jnp.float32), lh], 0)
    pz = jnp.einsum("ts,kj->tksj", eye_t, lz6).reshape(T * 6, T * F_OUT)
    ph = jnp.einsum("ts,kj->tksj", eye_t, lh6).reshape(T * 6, T * F_OUT)
    p = jnp.concatenate([pz, ph], axis=1)
    p = jnp.pad(p, ((0, D - 72), (0, D - 72)))

    bz = conv_z_b @ lz + lin_z_b
    bh = conv_h_b @ lh + lin_h_b
    bvec = jnp.concatenate([jnp.tile(bz, T), jnp.tile(bh, T),
                            jnp.zeros((D - 72,), jnp.float32)])[None, :]

    probs = jax.nn.softmax(attention)
    q = (probs[:, None, None] * jnp.eye(F_OUT, dtype=jnp.float32)[None]
         ).reshape(T * F_OUT, F_OUT)

    xf = x.reshape(N, F_IN * T)
    src3 = edge_index[0].reshape(NW, NCH, C)
    dst3 = edge_index[1].reshape(NW, NCH, C)
    zeros = jnp.zeros((N, D), jnp.float32)

    deg_kernel, edge_kernel = _sc_kernels()
    degp = deg_kernel(edge_index[1])
    degt = degp.T
    xws = _xws_call(xf, wb, degt, bn=1000)
    s2 = edge_kernel(src3, dst3, xws, zeros)
    return _post_call(s2, xws, degt, p, bvec, q, out_w, out_b[None, :],
                      bn=1000)
